# baseline (speedup 1.0000x reference)
import math
import numpy as np
import jax
import jax.numpy as jnp
from jax import lax
from jax.experimental import pallas as pl
from jax.experimental.pallas import tpu as pltpu
from jax.experimental.pallas import tpu_sc as plsc

IMG_H, IMG_W = 360, 640
S = 72
STRIDE = 8
FMAP_H, FMAP_W = IMG_H // STRIDE, IMG_W // STRIDE
IN_CH = 256
FEAT_CH = 64
B = 8
NOUT = 80
HP = FMAP_H + 1
ROWS_PER_B = HP * FMAP_W
_LEFT = [72., 60., 49., 39., 30., 22.]
_RIGHT = [108., 120., 131., 141., 150., 158.]
_BOTTOM = [165., 150., 141., 131., 120., 108., 100., 90., 80., 72., 60., 49., 39., 30., 15.]


def _mk_anchor(start, angle_deg, cut):
    n = FMAP_H if cut else S
    anchor_ys = np.linspace(1.0, 0.0, n, dtype=np.float64)
    a = np.zeros(2 + 2 + 1 + n, dtype=np.float32)
    angle = angle_deg * math.pi / 180.0
    sx, sy = start
    a[2] = 1.0 - sy
    a[3] = sx
    a[5:] = (sx + (1.0 - anchor_ys - 1.0 + sy) / math.tan(angle)) * IMG_W
    return a


def _mk_side(angles, nb, x=None, y=None):
    if x is None:
        starts = [(vx, y) for vx in np.linspace(1.0, 0.0, nb)]
    else:
        starts = [(x, vy) for vy in np.linspace(1.0, 0.0, nb)]
    anchors, cuts = [], []
    for s in starts:
        for ang in angles:
            anchors.append(_mk_anchor(s, ang, False))
            cuts.append(_mk_anchor(s, ang, True))
    return np.stack(anchors), np.stack(cuts)


def _geometry():
    la, lc = _mk_side(_LEFT, 72, x=0.0)
    ra, rc = _mk_side(_RIGHT, 72, x=1.0)
    ba, bc = _mk_side(_BOTTOM, 128, y=1.0)
    anchors = np.concatenate([la, ba, ra], axis=0)
    cuts = np.concatenate([lc, bc, rc], axis=0)
    xs_uncl = np.flip(np.round(cuts[:, 5:] / STRIDE), axis=1).astype(np.int64)
    cut_xs = np.clip(xs_uncl, 0, FMAP_W - 1)
    invalid = (xs_uncl < 0) | (xs_uncl > FMAP_W)
    return anchors, cut_xs, invalid


_ANCHORS_NP, _CUT_XS_NP, _INVALID_NP = _geometry()
N_ANCH = _ANCHORS_NP.shape[0]
TOTAL_ROWS = B * N_ANCH

NTILES = 32
TPB = 4
PT = N_ANCH // TPB
HPG = 5
NG_SC = 1
H_SC = NG_SC * HPG
N_TC = FMAP_H - H_SC
SLAB_R = HPG * FMAP_W
ZROW = SLAB_R
OG = 44
ACC_R = OG * 16
KU = 8
NOUTP = 81
ANCHOR_BASE = B * ROWS_PER_B
N_ANCH_PAD = ACC_R * TPB
TABLE_ROWS = ANCHOR_BASE


def _build_idx():
    idx = np.full((TPB, NG_SC, OG, HPG, 16), ZROW, dtype=np.int32)
    for tm in range(TPB):
        for g in range(NG_SC):
            for og in range(OG):
                p = tm * PT + og * 16 + np.arange(16)
                live = p < (tm + 1) * PT
                pc = np.minimum(p, N_ANCH - 1)
                for hl in range(HPG):
                    h = g * HPG + hl
                    r = hl * FMAP_W + _CUT_XS_NP[pc, h]
                    r = np.where(live & ~_INVALID_NP[pc, h], r, ZROW)
                    idx[tm, g, og, hl] = r
    return idx


_IDX_NP = _build_idx()

_ANCH_TILED_NP = np.zeros((TPB, ACC_R, NOUT), dtype=np.float32)
for _tm in range(TPB):
    _lo = _tm * PT
    _n = min(ACC_R, N_ANCH - _lo)
    _ANCH_TILED_NP[_tm, :_n, 2:77] = _ANCHORS_NP[_lo:_lo + _n, 2:]

_G_NP = np.zeros((N_TC * FMAP_W, N_ANCH_PAD), dtype=np.float32)
for _j in range(N_TC):
    _h = H_SC + _j
    _p = np.nonzero(~_INVALID_NP[:, _h])[0]
    _G_NP[_j * FMAP_W + _CUT_XS_NP[_p, _h], _p] = 1.0


def _project_body(xt_ref, wr_ref, wct_ref, bconv_ref, out_ref, outt_ref):
    wr = wr_ref[0]
    wcomb = jnp.dot(wct_ref[...], wr,
                    preferred_element_type=jnp.float32)
    xb = xt_ref[0].reshape(B * FMAP_W, IN_CH)
    m = jnp.dot(xb, wcomb, preferred_element_type=jnp.float32)
    wb = jnp.dot(bconv_ref[...], wr, preferred_element_type=jnp.float32)
    m = m + wb
    m = m.reshape(B, 1, FMAP_W, NOUT)
    out_ref[...] = m
    outt_ref[...] = m.transpose(1, 0, 3, 2)


def _project(xt, wr, wct, bconv):
    return pl.pallas_call(
        _project_body,
        grid=(HP,),
        in_specs=[
            pl.BlockSpec((1, B, FMAP_W, IN_CH),
                         lambda h: (jnp.minimum(h, FMAP_H - 1), 0, 0, 0)),
            pl.BlockSpec((1, FEAT_CH, NOUT), lambda h: (h, 0, 0)),
            pl.BlockSpec((IN_CH, FEAT_CH), lambda h: (0, 0)),
            pl.BlockSpec((1, FEAT_CH), lambda h: (0, 0)),
        ],
        out_specs=[
            pl.BlockSpec((B, 1, FMAP_W, NOUT), lambda h: (0, h, 0, 0)),
            pl.BlockSpec((1, B, NOUT, FMAP_W), lambda h: (h, 0, 0, 0)),
        ],
        out_shape=[
            jax.ShapeDtypeStruct((B, HP, FMAP_W, NOUT), jnp.float32),
            jax.ShapeDtypeStruct((HP, B, NOUT, FMAP_W), jnp.float32),
        ],
    )(xt, wr, wct, bconv)


def _gather_body(table_hbm, extra_hbm, idx_hbm, out_hbm, idx_v, slab_v, acc_v,
                 sem):
    t = lax.axis_index("s") * 2 + lax.axis_index("c")
    b = t // TPB
    tm = t - b * TPB
    for k in range(NOUT // 16):
        slab_v[ZROW, pl.ds(k * 16, 16)] = jnp.zeros((16,), jnp.float32)
    pltpu.sync_copy(extra_hbm.at[tm], acc_v)

    for g in range(NG_SC):
        pltpu.sync_copy(idx_hbm.at[tm, g], idx_v)
        pltpu.sync_copy(
            table_hbm.at[pl.ds((b * NG_SC + g) * SLAB_R, SLAB_R)],
            slab_v.at[pl.ds(0, SLAB_R), pl.ds(0, NOUT)])

        def og_step(og, carry):
            rows = [idx_v[og, hl] for hl in range(HPG)]
            a16 = og * 16

            def col_step(kq, carry2):
                for u in range(KU):
                    k = kq * KU + u
                    kvec = jnp.full((16,), k, jnp.int32)
                    vals = [acc_v[k, pl.ds(a16, 16)]]
                    vals += [plsc.load_gather(slab_v, [rows[hl], kvec])
                             for hl in range(HPG)]
                    while len(vals) > 1:
                        nxt = [vals[i] + vals[i + 1]
                               for i in range(0, len(vals) - 1, 2)]
                        if len(vals) % 2:
                            nxt.append(vals[-1])
                        vals = nxt
                    acc_v[k, pl.ds(a16, 16)] = vals[0]
                return carry2

            lax.fori_loop(0, NOUT // KU, col_step, 0)
            return carry

        lax.fori_loop(0, OG, og_step, 0)

    pltpu.sync_copy(acc_v.at[:, pl.ds(0, PT)],
                    out_hbm.at[b, :, pl.ds(tm * PT, PT)])


def _gather(table, extra_t, idx):
    mesh = plsc.VectorSubcoreMesh(core_axis_name="c", subcore_axis_name="s")
    f = pl.kernel(
        _gather_body,
        out_type=jax.ShapeDtypeStruct((B, NOUT, N_ANCH), jnp.float32),
        mesh=mesh,
        scratch_types=[
            pltpu.VMEM((OG, HPG, 16), jnp.int32),
            pltpu.VMEM((SLAB_R + 1, NOUTP), jnp.float32),
            pltpu.VMEM((NOUT, ACC_R), jnp.float32),
            pltpu.SemaphoreType.DMA,
        ],
        compiler_params=pltpu.CompilerParams(use_tc_tiling_on_sc=False,
                                             needs_layout_passes=False),
    )
    return f(table, extra_t, idx)


NBLK = N_ANCH_PAD // 2


def _combine_body(a_ref, g_ref, out_ref):
    out_ref[0] = jnp.dot(a_ref[0], g_ref[...],
                         preferred_element_type=jnp.float32)


def _combine(a_mat, g):
    return pl.pallas_call(
        _combine_body,
        grid=(N_ANCH_PAD // NBLK, B),
        in_specs=[
            pl.BlockSpec((1, NOUT, N_TC * FMAP_W), lambda nb, b: (b, 0, 0)),
            pl.BlockSpec((N_TC * FMAP_W, NBLK), lambda nb, b: (0, nb)),
        ],
        out_specs=pl.BlockSpec((1, NOUT, NBLK), lambda nb, b: (b, 0, nb)),
        out_shape=jax.ShapeDtypeStruct((B, NOUT, N_ANCH_PAD), jnp.float32),
    )(a_mat, g)


def _final_body(sc_ref, comb_ref, out_ref):
    s = sc_ref[0] + comb_ref[0, :, :N_ANCH]
    out_ref[0] = s.T[:, :77]


def _final(sc, comb):
    return pl.pallas_call(
        _final_body,
        grid=(B,),
        in_specs=[
            pl.BlockSpec((1, NOUT, N_ANCH), lambda b: (b, 0, 0)),
            pl.BlockSpec((1, NOUT, N_ANCH_PAD), lambda b: (b, 0, 0)),
        ],
        out_specs=pl.BlockSpec((1, N_ANCH, 77), lambda b: (b, 0, 0)),
        out_shape=jax.ShapeDtypeStruct((B, N_ANCH, 77), jnp.float32),
    )(sc, comb)


def kernel(x, W_conv, b_conv, W_cls, b_cls, W_reg, b_reg):
    feat_dim = FEAT_CH * FMAP_H
    zero2 = jnp.zeros((2, feat_dim), jnp.float32)
    zero3 = jnp.zeros((3, feat_dim), jnp.float32)
    wfull = jnp.concatenate([W_cls, zero2, W_reg, zero3], axis=0)
    wr = wfull.reshape(NOUT, FEAT_CH, FMAP_H).transpose(2, 1, 0)
    wr = jnp.concatenate([wr, jnp.zeros((1, FEAT_CH, NOUT), jnp.float32)], 0)
    wct = W_conv[:, :, 0, 0].T
    xt = x.transpose(2, 0, 3, 1)

    m2, m2t = _project(xt, wr, wct, b_conv.reshape(1, FEAT_CH))

    bias = jnp.concatenate([b_cls, jnp.zeros((2,), jnp.float32),
                            b_reg, jnp.zeros((3,), jnp.float32)])
    extra = jnp.asarray(_ANCH_TILED_NP) + bias[None, None, :]
    extra_t = extra.transpose(0, 2, 1)
    table = m2[:, :H_SC].reshape(B * H_SC * FMAP_W, NOUT)

    sc = _gather(table, extra_t, jnp.asarray(_IDX_NP))
    a_mat = (m2t[H_SC:FMAP_H].transpose(1, 2, 0, 3)
             .reshape(B, NOUT, N_TC * FMAP_W)
             .astype(jnp.bfloat16))
    comb = _combine(a_mat, jnp.asarray(_G_NP, dtype=jnp.bfloat16))
    return _final(sc, comb)


# speedup vs baseline: 1.0173x; 1.0173x over previous
import math
import numpy as np
import jax
import jax.numpy as jnp
from jax import lax
from jax.experimental import pallas as pl
from jax.experimental.pallas import tpu as pltpu
from jax.experimental.pallas import tpu_sc as plsc

IMG_H, IMG_W = 360, 640
S = 72
STRIDE = 8
FMAP_H, FMAP_W = IMG_H // STRIDE, IMG_W // STRIDE
IN_CH = 256
FEAT_CH = 64
B = 8
NOUT = 80
HP = FMAP_H + 1
ROWS_PER_B = HP * FMAP_W
_LEFT = [72., 60., 49., 39., 30., 22.]
_RIGHT = [108., 120., 131., 141., 150., 158.]
_BOTTOM = [165., 150., 141., 131., 120., 108., 100., 90., 80., 72., 60., 49., 39., 30., 15.]


def _mk_anchor(start, angle_deg, cut):
    n = FMAP_H if cut else S
    anchor_ys = np.linspace(1.0, 0.0, n, dtype=np.float64)
    a = np.zeros(2 + 2 + 1 + n, dtype=np.float32)
    angle = angle_deg * math.pi / 180.0
    sx, sy = start
    a[2] = 1.0 - sy
    a[3] = sx
    a[5:] = (sx + (1.0 - anchor_ys - 1.0 + sy) / math.tan(angle)) * IMG_W
    return a


def _mk_side(angles, nb, x=None, y=None):
    if x is None:
        starts = [(vx, y) for vx in np.linspace(1.0, 0.0, nb)]
    else:
        starts = [(x, vy) for vy in np.linspace(1.0, 0.0, nb)]
    anchors, cuts = [], []
    for s in starts:
        for ang in angles:
            anchors.append(_mk_anchor(s, ang, False))
            cuts.append(_mk_anchor(s, ang, True))
    return np.stack(anchors), np.stack(cuts)


def _geometry():
    la, lc = _mk_side(_LEFT, 72, x=0.0)
    ra, rc = _mk_side(_RIGHT, 72, x=1.0)
    ba, bc = _mk_side(_BOTTOM, 128, y=1.0)
    anchors = np.concatenate([la, ba, ra], axis=0)
    cuts = np.concatenate([lc, bc, rc], axis=0)
    xs_uncl = np.flip(np.round(cuts[:, 5:] / STRIDE), axis=1).astype(np.int64)
    cut_xs = np.clip(xs_uncl, 0, FMAP_W - 1)
    invalid = (xs_uncl < 0) | (xs_uncl > FMAP_W)
    return anchors, cut_xs, invalid


_ANCHORS_NP, _CUT_XS_NP, _INVALID_NP = _geometry()
N_ANCH = _ANCHORS_NP.shape[0]
TOTAL_ROWS = B * N_ANCH

NTILES = 32
TPB = 4
PT = N_ANCH // TPB
HPG = 5
NG_SC = 1
H_SC = NG_SC * HPG
N_TC = FMAP_H - H_SC
SLAB_R = HPG * FMAP_W
ZROW = SLAB_R
OG = 44
ACC_R = OG * 16
KU = 8
NOUTP = 81
ANCHOR_BASE = B * ROWS_PER_B
N_ANCH_PAD = ACC_R * TPB
TABLE_ROWS = ANCHOR_BASE


def _build_idx():
    idx = np.full((TPB, NG_SC, OG, HPG, 16), ZROW, dtype=np.int32)
    for tm in range(TPB):
        for g in range(NG_SC):
            for og in range(OG):
                p = tm * PT + og * 16 + np.arange(16)
                live = p < (tm + 1) * PT
                pc = np.minimum(p, N_ANCH - 1)
                for hl in range(HPG):
                    h = g * HPG + hl
                    r = hl * FMAP_W + _CUT_XS_NP[pc, h]
                    r = np.where(live & ~_INVALID_NP[pc, h], r, ZROW)
                    idx[tm, g, og, hl] = r
    return idx


_IDX_NP = _build_idx()

_ANCH_TILED_NP = np.zeros((TPB, ACC_R, NOUT), dtype=np.float32)
for _tm in range(TPB):
    _lo = _tm * PT
    _n = min(ACC_R, N_ANCH - _lo)
    _ANCH_TILED_NP[_tm, :_n, 2:77] = _ANCHORS_NP[_lo:_lo + _n, 2:]

_G_NP = np.zeros((N_TC * FMAP_W, N_ANCH_PAD), dtype=np.float32)
for _j in range(N_TC):
    _h = H_SC + _j
    _p = np.nonzero(~_INVALID_NP[:, _h])[0]
    _G_NP[_j * FMAP_W + _CUT_XS_NP[_p, _h], _p] = 1.0


def _project_body(xt_ref, wr_ref, wct_ref, bconv_ref, out_ref, outt_ref):
    wr = wr_ref[0]
    wcomb = jnp.dot(wct_ref[...], wr,
                    preferred_element_type=jnp.float32)
    xb = xt_ref[0].reshape(B * FMAP_W, IN_CH)
    m = jnp.dot(xb, wcomb, preferred_element_type=jnp.float32)
    wb = jnp.dot(bconv_ref[...], wr, preferred_element_type=jnp.float32)
    m = m + wb
    m = m.reshape(B, 1, FMAP_W, NOUT)
    out_ref[...] = m
    outt_ref[...] = m.transpose(1, 0, 3, 2)


def _project(xt, wr, wct, bconv):
    return pl.pallas_call(
        _project_body,
        grid=(HP,),
        in_specs=[
            pl.BlockSpec((1, B, FMAP_W, IN_CH),
                         lambda h: (jnp.minimum(h, FMAP_H - 1), 0, 0, 0)),
            pl.BlockSpec((1, FEAT_CH, NOUT), lambda h: (h, 0, 0)),
            pl.BlockSpec((IN_CH, FEAT_CH), lambda h: (0, 0)),
            pl.BlockSpec((1, FEAT_CH), lambda h: (0, 0)),
        ],
        out_specs=[
            pl.BlockSpec((B, 1, FMAP_W, NOUT), lambda h: (0, h, 0, 0)),
            pl.BlockSpec((1, B, NOUT, FMAP_W), lambda h: (h, 0, 0, 0)),
        ],
        out_shape=[
            jax.ShapeDtypeStruct((B, HP, FMAP_W, NOUT), jnp.float32),
            jax.ShapeDtypeStruct((HP, B, NOUT, FMAP_W), jnp.float32),
        ],
    )(xt, wr, wct, bconv)


def _gather_body(table_hbm, extra_hbm, idx_hbm, out_hbm, idx_v, slab_v, acc_v,
                 sem):
    t = lax.axis_index("s") * 2 + lax.axis_index("c")
    b = t // TPB
    tm = t - b * TPB
    for k in range(NOUT // 16):
        slab_v[ZROW, pl.ds(k * 16, 16)] = jnp.zeros((16,), jnp.float32)
    pltpu.sync_copy(extra_hbm.at[tm], acc_v)

    for g in range(NG_SC):
        pltpu.sync_copy(idx_hbm.at[tm, g], idx_v)
        pltpu.sync_copy(
            table_hbm.at[pl.ds((b * NG_SC + g) * SLAB_R, SLAB_R)],
            slab_v.at[pl.ds(0, SLAB_R), pl.ds(0, NOUT)])

        def og_step(og, carry):
            rows = [idx_v[og, hl] for hl in range(HPG)]
            a16 = og * 16

            def col_step(kq, carry2):
                for u in range(KU):
                    k = kq * KU + u
                    kvec = jnp.full((16,), k, jnp.int32)
                    vals = [acc_v[k, pl.ds(a16, 16)]]
                    vals += [plsc.load_gather(slab_v, [rows[hl], kvec])
                             for hl in range(HPG)]
                    while len(vals) > 1:
                        nxt = [vals[i] + vals[i + 1]
                               for i in range(0, len(vals) - 1, 2)]
                        if len(vals) % 2:
                            nxt.append(vals[-1])
                        vals = nxt
                    acc_v[k, pl.ds(a16, 16)] = vals[0]
                return carry2

            lax.fori_loop(0, NOUT // KU, col_step, 0)
            return carry

        lax.fori_loop(0, OG, og_step, 0)

    pltpu.sync_copy(acc_v.at[:, pl.ds(0, PT)],
                    out_hbm.at[b, :, pl.ds(tm * PT, PT)])


def _gather(table, extra_t, idx):
    mesh = plsc.VectorSubcoreMesh(core_axis_name="c", subcore_axis_name="s")
    f = pl.kernel(
        _gather_body,
        out_type=jax.ShapeDtypeStruct((B, NOUT, N_ANCH), jnp.float32),
        mesh=mesh,
        scratch_types=[
            pltpu.VMEM((OG, HPG, 16), jnp.int32),
            pltpu.VMEM((SLAB_R + 1, NOUTP), jnp.float32),
            pltpu.VMEM((NOUT, ACC_R), jnp.float32),
            pltpu.SemaphoreType.DMA,
        ],
        compiler_params=pltpu.CompilerParams(use_tc_tiling_on_sc=False,
                                             needs_layout_passes=False),
    )
    return f(table, extra_t, idx)


NBLK = N_ANCH_PAD // 2


def _combine_body(a_ref, g_ref, out_ref):
    out_ref[0] = jnp.dot(a_ref[0], g_ref[...],
                         preferred_element_type=jnp.float32)


def _combine(a_mat, g):
    return pl.pallas_call(
        _combine_body,
        grid=(N_ANCH_PAD // NBLK, B),
        in_specs=[
            pl.BlockSpec((1, NOUT, N_TC * FMAP_W), lambda nb, b: (b, 0, 0)),
            pl.BlockSpec((N_TC * FMAP_W, NBLK), lambda nb, b: (0, nb)),
        ],
        out_specs=pl.BlockSpec((1, NOUT, NBLK), lambda nb, b: (b, 0, nb)),
        out_shape=jax.ShapeDtypeStruct((B, NOUT, N_ANCH_PAD), jnp.float32),
    )(a_mat, g)


def _final_body(sc_ref, comb_ref, out_ref):
    out_ref[...] = sc_ref[...] + comb_ref[:, :, :N_ANCH]


def _final(sc, comb):
    return pl.pallas_call(
        _final_body,
        grid=(B,),
        in_specs=[
            pl.BlockSpec((1, NOUT, N_ANCH), lambda b: (b, 0, 0)),
            pl.BlockSpec((1, NOUT, N_ANCH_PAD), lambda b: (b, 0, 0)),
        ],
        out_specs=pl.BlockSpec((1, NOUT, N_ANCH), lambda b: (b, 0, 0)),
        out_shape=jax.ShapeDtypeStruct((B, NOUT, N_ANCH), jnp.float32),
    )(sc, comb)


def kernel(x, W_conv, b_conv, W_cls, b_cls, W_reg, b_reg):
    feat_dim = FEAT_CH * FMAP_H
    zero2 = jnp.zeros((2, feat_dim), jnp.float32)
    zero3 = jnp.zeros((3, feat_dim), jnp.float32)
    wfull = jnp.concatenate([W_cls, zero2, W_reg, zero3], axis=0)
    wr = wfull.reshape(NOUT, FEAT_CH, FMAP_H).transpose(2, 1, 0)
    wr = jnp.concatenate([wr, jnp.zeros((1, FEAT_CH, NOUT), jnp.float32)], 0)
    wct = W_conv[:, :, 0, 0].T
    xt = x.transpose(2, 0, 3, 1)

    m2, m2t = _project(xt, wr, wct, b_conv.reshape(1, FEAT_CH))

    bias = jnp.concatenate([b_cls, jnp.zeros((2,), jnp.float32),
                            b_reg, jnp.zeros((3,), jnp.float32)])
    extra = jnp.asarray(_ANCH_TILED_NP) + bias[None, None, :]
    extra_t = extra.transpose(0, 2, 1)
    table = m2[:, :H_SC].reshape(B * H_SC * FMAP_W, NOUT)

    sc = _gather(table, extra_t, jnp.asarray(_IDX_NP))
    a_mat = (m2t[H_SC:FMAP_H].transpose(1, 2, 0, 3)
             .reshape(B, NOUT, N_TC * FMAP_W)
             .astype(jnp.bfloat16))
    comb = _combine(a_mat, jnp.asarray(_G_NP, dtype=jnp.bfloat16))
    out = _final(sc, comb)
    return out.transpose(0, 2, 1)[:, :, :77]
